# Initial kernel scaffold; baseline (speedup 1.0000x reference)
import jax
import jax.numpy as jnp
from jax.experimental import pallas as pl


def kernel(SidebarAssetName, SidebarContinuous, buildable_embedding_weight):
    raise NotImplementedError("write your pallas kernel here")


# speedup vs baseline: 2.7793x; 2.7793x over previous
import functools

import jax
import jax.numpy as jnp
from jax import lax
from jax.experimental import pallas as pl
from jax.experimental.pallas import tpu as pltpu
from jax.experimental.pallas import tpu_sc as plsc

NUM_EMBEDDINGS = 1000
EMBED_DIM = 7
CONT_DIM = 6
OUT_DIM = EMBED_DIM + CONT_DIM

NC = 2
NS = 16
NW = NC * NS
LANES = 16

B = 16384
L = 200
N = B * L
N_PER_W = N // NW
C = 2048
N_CHUNKS = N_PER_W // C
GROUPS = C // LANES


def _iota16():
  return lax.iota(jnp.int32, LANES)


def _sc_body(idx_hbm, cont_hbm, table_hbm, out_hbm, table_v, idx_v, cont_v,
             out_v):
  wid = lax.axis_index("s") * NC + lax.axis_index("c")
  base0 = wid * N_PER_W

  pltpu.sync_copy(table_hbm, table_v)

  iota = _iota16()
  emb_pos = [iota * OUT_DIM + c for c in range(EMBED_DIM)]
  cont_pos = []
  for k in range(CONT_DIM):
    e = k * LANES + iota
    cont_pos.append((e // CONT_DIM) * OUT_DIM + e % CONT_DIM + EMBED_DIM)

  def chunk_body(ci, _):
    base = base0 + ci * C
    pltpu.sync_copy(idx_hbm.at[pl.ds(base, C)], idx_v)
    pltpu.sync_copy(cont_hbm.at[pl.ds(base * CONT_DIM, C * CONT_DIM)], cont_v)

    def group_body(g, _):
      r0 = g * LANES
      out_base = r0 * OUT_DIM
      idxv = idx_v[pl.ds(r0, LANES)]
      src_base = idxv * EMBED_DIM
      for c in range(EMBED_DIM):
        vals = plsc.load_gather(table_v, [src_base + c])
        plsc.store_scatter(out_v, [out_base + emb_pos[c]], vals)
      cbase = r0 * CONT_DIM
      for k in range(CONT_DIM):
        vals = cont_v[pl.ds(cbase + k * LANES, LANES)]
        plsc.store_scatter(out_v, [out_base + cont_pos[k]], vals)
      return 0

    lax.fori_loop(0, GROUPS, group_body, 0)
    pltpu.sync_copy(out_v, out_hbm.at[pl.ds(base * OUT_DIM, C * OUT_DIM)])
    return 0

  lax.fori_loop(0, N_CHUNKS, chunk_body, 0)


@jax.jit
def _run(idx_flat, cont_flat, table_flat):
  mesh = plsc.VectorSubcoreMesh(
      core_axis_name="c", subcore_axis_name="s", num_cores=NC,
      num_subcores=NS)
  f = pl.kernel(
      _sc_body,
      out_type=jax.ShapeDtypeStruct((N * OUT_DIM,), jnp.float32),
      mesh=mesh,
      compiler_params=pltpu.CompilerParams(needs_layout_passes=False),
      scratch_types=[
          pltpu.VMEM((NUM_EMBEDDINGS * EMBED_DIM,), jnp.float32),
          pltpu.VMEM((C,), jnp.int32),
          pltpu.VMEM((C * CONT_DIM,), jnp.float32),
          pltpu.VMEM((C * OUT_DIM,), jnp.float32),
      ],
  )
  return f(idx_flat, cont_flat, table_flat)


def kernel(SidebarAssetName, SidebarContinuous, buildable_embedding_weight):
  idx_flat = SidebarAssetName.reshape(N).astype(jnp.int32)
  cont_flat = SidebarContinuous.reshape(N * CONT_DIM)
  table_flat = buildable_embedding_weight.reshape(NUM_EMBEDDINGS * EMBED_DIM)
  out = _run(idx_flat, cont_flat, table_flat)
  return out.reshape(B, L, OUT_DIM)


# speedup vs baseline: 29.6271x; 10.6600x over previous
import jax
import jax.numpy as jnp
from jax import lax
from jax.experimental import pallas as pl
from jax.experimental.pallas import tpu as pltpu
from jax.experimental.pallas import tpu_sc as plsc

NUM_EMBEDDINGS = 1000
EMBED_DIM = 7
CONT_DIM = 6
OUT_DIM = EMBED_DIM + CONT_DIM

NC = 2
NS = 16
NW = NC * NS
LANES = 16

B = 16384
L = 200
TAB_STRIDE = 1024

BW = B // NW
NLB = L // 8
XW = BW // LANES


def _sc_body(idx_hbm, cont_hbm, tab_hbm, out_hbm, tab_v, idx_v, buf_v,
             emb_v):
  wid = lax.axis_index("s") * NC + lax.axis_index("c")
  b0 = wid * BW

  pltpu.sync_copy(tab_hbm, tab_v)

  def lblock(li, _):
    l0 = li * 8
    pltpu.sync_copy(idx_hbm.at[pl.ds(l0, 8), pl.ds(b0, BW)], idx_v)

    for j in range(CONT_DIM):
      pltpu.sync_copy(cont_hbm.at[j, pl.ds(l0, 8), pl.ds(b0, BW)], buf_v)
      pltpu.sync_copy(buf_v,
                      out_hbm.at[EMBED_DIM + j, pl.ds(l0, 8), pl.ds(b0, BW)])

    def gcol(x, _):
      xoff = x * LANES
      for r in range(8):
        iv = idx_v[r, pl.ds(xoff, LANES)]
        for c in range(EMBED_DIM):
          vals = plsc.load_gather(tab_v, [iv + (c * TAB_STRIDE)])
          emb_v[c, r, pl.ds(xoff, LANES)] = vals
      return 0

    lax.fori_loop(0, XW, gcol, 0)
    for c in range(EMBED_DIM):
      pltpu.sync_copy(emb_v.at[c],
                      out_hbm.at[c, pl.ds(l0, 8), pl.ds(b0, BW)])
    return 0

  lax.fori_loop(0, NLB, lblock, 0)


@jax.jit
def _run(idx_t, cont_t, tab_cols):
  mesh = plsc.VectorSubcoreMesh(
      core_axis_name="c", subcore_axis_name="s", num_cores=NC,
      num_subcores=NS)
  f = pl.kernel(
      _sc_body,
      out_type=jax.ShapeDtypeStruct((OUT_DIM, L, B), jnp.float32),
      mesh=mesh,
      compiler_params=pltpu.CompilerParams(
          needs_layout_passes=False, use_tc_tiling_on_sc=True),
      scratch_types=[
          pltpu.VMEM((EMBED_DIM * TAB_STRIDE,), jnp.float32),
          pltpu.VMEM((8, BW), jnp.int32),
          pltpu.VMEM((8, BW), jnp.float32),
          pltpu.VMEM((EMBED_DIM, 8, BW), jnp.float32),
      ],
  )
  return f(idx_t, cont_t, tab_cols)


def kernel(SidebarAssetName, SidebarContinuous, buildable_embedding_weight):
  idx_t = jnp.transpose(SidebarAssetName.astype(jnp.int32), (1, 0))
  cont_t = jnp.transpose(SidebarContinuous, (2, 1, 0))
  tab_cols = jnp.zeros((EMBED_DIM, TAB_STRIDE), jnp.float32)
  tab_cols = tab_cols.at[:, :NUM_EMBEDDINGS].set(
      buildable_embedding_weight.T).reshape(EMBED_DIM * TAB_STRIDE)
  out = _run(idx_t, cont_t, tab_cols)
  return jnp.transpose(out, (2, 1, 0))
